# baseline (speedup 1.0000x reference)
import functools

import jax
import jax.numpy as jnp
from jax import lax
from jax.experimental import pallas as pl
from jax.experimental.pallas import tpu as pltpu
from jax.experimental.pallas import tpu_sc as plsc

PAD = 0.0
B, L = 16384, 200
W = L + 1
NW = 32
RPW = B // NW
NG = RPW // 16

_CHUNK_OFFS = tuple(range(0, L - 16, 16)) + (L - 16,)

_mesh = plsc.VectorSubcoreMesh(core_axis_name="c", subcore_axis_name="s")


@functools.partial(
    pl.kernel,
    mesh=_mesh,
    out_type=[
        jax.ShapeDtypeStruct((B, W), jnp.float32),
        jax.ShapeDtypeStruct((B,), jnp.int32),
    ],
    scratch_types=[
        pltpu.VMEM((RPW, L), jnp.float32),
        pltpu.VMEM((RPW,), jnp.int32),
        pltpu.VMEM((RPW, 1), jnp.float32),
        pltpu.SemaphoreType.DMA,
        pltpu.SemaphoreType.DMA,
    ],
    compiler_params=pltpu.CompilerParams(use_tc_tiling_on_sc=False),
)
def _sc_trim(col_hbm, zcol_hbm, dense_hbm, rl_hbm, buf, lens_v, mz, sem, sem2):
    wid = lax.axis_index("s") * 2 + lax.axis_index("c")
    base = wid * RPW
    iota16 = lax.iota(jnp.int32, 16)

    pltpu.sync_copy(zcol_hbm.at[pl.ds(0, RPW), :], mz)
    marker_copy = pltpu.async_copy(
        mz, dense_hbm.at[pl.ds(base, RPW), pl.ds(L, 1)], sem2
    )

    pltpu.sync_copy(col_hbm.at[pl.ds(base, RPW), :], buf)

    out_copy = pltpu.async_copy(
        buf, dense_hbm.at[pl.ds(base, RPW), pl.ds(0, L)], sem
    )

    pos_vecs = [iota16 + (off + 1) for off in _CHUNK_OFFS]
    rot_idx = [(iota16 + s) % 16 for s in (8, 4, 2, 1)]

    def group_body(g, carry):
        lenvec = jnp.zeros((16,), jnp.int32)
        for rr in range(16):
            r = g * 16 + rr
            acc = jnp.zeros((16,), jnp.int32)
            for off, pos in zip(_CHUNK_OFFS, pos_vecs):
                x = buf[r, pl.ds(off, 16)]
                acc = jnp.where(x != PAD, pos, acc)
            for idx in rot_idx:
                acc = jnp.maximum(acc, acc.at[idx].get(mode="promise_in_bounds"))
            lenvec = jnp.where(iota16 == rr, acc + 1, lenvec)
        lens_v[pl.ds(g * 16, 16)] = lenvec
        return carry

    lax.fori_loop(0, NG, group_body, 0)

    pltpu.sync_copy(lens_v, rl_hbm.at[pl.ds(base, RPW)])
    marker_copy.wait()
    out_copy.wait()


@jax.jit
def kernel(column):
    zcol = jnp.zeros((RPW, 1), jnp.float32)
    dense, row_lengths = _sc_trim(column, zcol)
    return dense, row_lengths


# speedup vs baseline: 1.2713x; 1.2713x over previous
import functools

import jax
import jax.numpy as jnp
from jax import lax
from jax.experimental import pallas as pl
from jax.experimental.pallas import tpu as pltpu
from jax.experimental.pallas import tpu_sc as plsc

PAD = 0.0
B, L = 16384, 200
W = L + 1
NW = 32
RPW = B // NW
NG = RPW // 16
BS = 4096

_CHUNK_OFFS = tuple(range(0, L - 16, 16)) + (L - 16,)

_mesh = plsc.VectorSubcoreMesh(core_axis_name="c", subcore_axis_name="s")


@functools.partial(
    pl.kernel,
    mesh=_mesh,
    out_type=jax.ShapeDtypeStruct((B,), jnp.int32),
    scratch_types=[
        pltpu.VMEM((RPW, L), jnp.float32),
        pltpu.VMEM((RPW,), jnp.int32),
    ],
    compiler_params=pltpu.CompilerParams(use_tc_tiling_on_sc=False),
)
def _sc_lengths(col_hbm, rl_hbm, buf, lens_v):
    wid = lax.axis_index("s") * 2 + lax.axis_index("c")
    base = wid * RPW
    iota16 = lax.iota(jnp.int32, 16)

    pltpu.sync_copy(col_hbm.at[pl.ds(base, RPW), :], buf)

    pos_vecs = [iota16 + (off + 1) for off in _CHUNK_OFFS]
    rot_idx = [(iota16 + s) % 16 for s in (8, 4, 2, 1)]

    def group_body(g, carry):
        lenvec = jnp.zeros((16,), jnp.int32)
        for rr in range(16):
            r = g * 16 + rr
            acc = jnp.zeros((16,), jnp.int32)
            for off, pos in zip(_CHUNK_OFFS, pos_vecs):
                x = buf[r, pl.ds(off, 16)]
                acc = jnp.where(x != PAD, pos, acc)
            for idx in rot_idx:
                acc = jnp.maximum(acc, acc.at[idx].get(mode="promise_in_bounds"))
            lenvec = jnp.where(iota16 == rr, acc + 1, lenvec)
        lens_v[pl.ds(g * 16, 16)] = lenvec
        return carry

    lax.fori_loop(0, NG, group_body, 0)

    pltpu.sync_copy(lens_v, rl_hbm.at[pl.ds(base, RPW)])


def _copy_block(x_ref, dense_ref):
    dense_ref[:, :L] = x_ref[...]
    dense_ref[:, L:] = jnp.zeros((BS, 1), jnp.float32)


def _tc_dense(column):
    return pl.pallas_call(
        _copy_block,
        grid=(B // BS,),
        in_specs=[pl.BlockSpec((BS, L), lambda i: (i, 0))],
        out_specs=pl.BlockSpec((BS, W), lambda i: (i, 0)),
        out_shape=jax.ShapeDtypeStruct((B, W), jnp.float32),
    )(column)


@jax.jit
def kernel(column):
    row_lengths = _sc_lengths(column)
    dense = _tc_dense(column)
    return dense, row_lengths


# speedup vs baseline: 2.3321x; 1.8345x over previous
import jax
import jax.numpy as jnp
from jax.experimental import pallas as pl

PAD = 0.0
B, L = 16384, 200
BS = 8192


def _trim_block(x_ref, dense_ref, len_ref):
    x = x_ref[...]
    dense_ref[:, :L] = x
    dense_ref[:, L:] = jnp.zeros((BS, 1), x.dtype)
    pos1 = jax.lax.broadcasted_iota(jnp.int32, (BS, L), 1) + 1
    lengths = jnp.max(jnp.where(x != PAD, pos1, 0), axis=1)
    len_ref[...] = lengths + 1


@jax.jit
def kernel(column):
    grid = (B // BS,)
    dense, row_lengths = pl.pallas_call(
        _trim_block,
        grid=grid,
        in_specs=[pl.BlockSpec((BS, L), lambda i: (i, 0))],
        out_specs=[
            pl.BlockSpec((BS, L + 1), lambda i: (i, 0)),
            pl.BlockSpec((BS,), lambda i: (i,)),
        ],
        out_shape=[
            jax.ShapeDtypeStruct((B, L + 1), column.dtype),
            jax.ShapeDtypeStruct((B,), jnp.int32),
        ],
    )(column)
    return dense, row_lengths
